# Initial kernel scaffold; baseline (speedup 1.0000x reference)
import jax
import jax.numpy as jnp
from jax.experimental import pallas as pl


def kernel(logit_previous, side_information, v, b, weights, boolean_converter, bias):
    raise NotImplementedError("write your pallas kernel here")


# speedup vs baseline: 13.7546x; 13.7546x over previous
import functools

import jax
import jax.numpy as jnp
from jax.experimental import pallas as pl
from jax.experimental.pallas import tpu as pltpu

N = 1024
I = 1024
S = 2048
C = 4
K = 2 ** C
B = 256

NB = 128


def _lv_block(v_ref, b_ref, w_ref, si_ref, lp_ref, out_ref):
    proj = jnp.dot(v_ref[:], si_ref[:], preferred_element_type=jnp.float32)
    c_of_row = jax.lax.broadcasted_iota(jnp.int32, (NB * C, 1), 0) % C
    pow2 = (1 << c_of_row).astype(jnp.float32)
    wb = jnp.where(proj > b_ref[:], pow2, 0.0)
    n_idx = jax.lax.broadcasted_iota(jnp.int32, (NB, NB * C), 0)
    r_idx = jax.lax.broadcasted_iota(jnp.int32, (NB, NB * C), 1)
    g4 = (r_idx // C == n_idx).astype(jnp.float32)
    ctx = jnp.dot(g4, wb, preferred_element_type=jnp.float32)

    m = jnp.dot(w_ref[:], lp_ref[:], preferred_element_type=jnp.float32)
    m3 = m.reshape(NB, K, B)
    kio = jax.lax.broadcasted_iota(jnp.int32, (1, K, 1), 1)
    ctx_i = ctx.astype(jnp.int32)
    sel = jnp.where(ctx_i[:, None, :] == kio, m3, 0.0)
    out_ref[:] = jnp.sum(sel, axis=1)


@functools.partial(jax.jit, static_argnames=())
def _layer_vec(lp, si, v2d, b2d, w2d):
    grid = (N // NB,)
    out = pl.pallas_call(
        _lv_block,
        grid=grid,
        in_specs=[
            pl.BlockSpec((NB * C, S), lambda i: (i, 0)),
            pl.BlockSpec((NB * C, 1), lambda i: (i, 0)),
            pl.BlockSpec((NB * K, I), lambda i: (i, 0)),
            pl.BlockSpec((S, B), lambda i: (0, 0)),
            pl.BlockSpec((I, B), lambda i: (0, 0)),
        ],
        out_specs=pl.BlockSpec((NB, B), lambda i: (i, 0)),
        out_shape=jax.ShapeDtypeStruct((N, B), jnp.float32),
    )(v2d, b2d, w2d, si, lp)
    return out


def kernel(logit_previous, side_information, v, b, weights, boolean_converter, bias):
    v2d = v.reshape(N * C, S)
    b2d = b.reshape(N * C, 1)
    w2d = weights.reshape(N * K, I)
    out = _layer_vec(logit_previous, side_information, v2d, b2d, w2d)
    out = out.at[0].set(bias)
    return out
